# Initial kernel scaffold; baseline (speedup 1.0000x reference)
import jax
import jax.numpy as jnp
from jax.experimental import pallas as pl


def kernel(x, codes, anchors, W_x, b_x, W_c, b_c, Wm1_0, bm1_0, Wm2_0, bm2_0, Wu1_0, bu1_0, Wu2_0, bu2_0, Wm1_1, bm1_1, Wm2_1, bm2_1, Wu1_1, bu1_1, Wu2_1, bu2_1, W_o, b_o):
    raise NotImplementedError("write your pallas kernel here")


# speedup vs baseline: 3.5027x; 3.5027x over previous
import jax
import jax.numpy as jnp
from jax.experimental import pallas as pl

TILE = 256
K_NEIGHBORS = 16
RADIUS2 = 16.0


def _decoder_kernel(x_ref, codes_ref, anchors_ref, W_x, b_x, W_c, b_c,
                    Wm1h_0, Wm1c_0, wd_0, bm1_0, Wm2_0, bm2_0,
                    Wu1h_0, Wu1a_0, bu1_0, Wu2_0, bu2_0,
                    Wm1h_1, Wm1c_1, wd_1, bm1_1, Wm2_1, bm2_1,
                    Wu1h_1, Wu1a_1, bu1_1, Wu2_1, bu2_1,
                    W_o, b_o, out_ref):
    x = x_ref[0]
    codes = codes_ref[0]
    anchors = anchors_ref[...]
    G = anchors.shape[0]

    h = jnp.tanh(jnp.dot(x, W_x[...], preferred_element_type=jnp.float32)
                 + b_x[...])
    c = jnp.tanh(jnp.dot(codes, W_c[...], preferred_element_type=jnp.float32)
                 + b_c[...])

    dx = x[:, 0:1] - anchors[:, 0][None, :]
    dy = x[:, 1:2] - anchors[:, 1][None, :]
    dz = x[:, 2:3] - anchors[:, 2][None, :]
    d2 = (dx * dx + dy * dy) + dz * dz

    a = d2[:, :, None]
    b = d2[:, None, :]
    gi = jax.lax.broadcasted_iota(jnp.int32, (1, G, G), 1)
    gj = jax.lax.broadcasted_iota(jnp.int32, (1, G, G), 2)
    closer = (b < a) | ((b == a) & (gj < gi))
    rank = jnp.sum(closer.astype(jnp.float32), axis=-1)
    select = ((rank < K_NEIGHBORS) & (d2 <= RADIUS2)).astype(jnp.float32)
    cnt = jnp.sum(select, axis=-1, keepdims=True)

    def layer(h, Wm1h, Wm1c, wd, bm1, Wm2, bm2, Wu1h, Wu1a, bu1, Wu2, bu2):
        hW = jnp.dot(h, Wm1h[...], preferred_element_type=jnp.float32) \
            + bm1[...]
        cW = jnp.dot(c, Wm1c[...], preferred_element_type=jnp.float32)
        pre = (hW[:, None, :] + cW[None, :, :]
               + d2[:, :, None] * wd[...][None, :, :])
        m = pre * jax.nn.sigmoid(pre)
        S = jnp.sum(m * select[:, :, None], axis=1)
        agg = jnp.dot(S, Wm2[...], preferred_element_type=jnp.float32) \
            + bm2[...] * cnt
        u = jax.nn.silu(jnp.dot(h, Wu1h[...], preferred_element_type=jnp.float32)
                        + jnp.dot(agg, Wu1a[...], preferred_element_type=jnp.float32)
                        + bu1[...])
        return h + jnp.dot(u, Wu2[...], preferred_element_type=jnp.float32) \
            + bu2[...]

    h = layer(h, Wm1h_0, Wm1c_0, wd_0, bm1_0, Wm2_0, bm2_0,
              Wu1h_0, Wu1a_0, bu1_0, Wu2_0, bu2_0)
    h = layer(h, Wm1h_1, Wm1c_1, wd_1, bm1_1, Wm2_1, bm2_1,
              Wu1h_1, Wu1a_1, bu1_1, Wu2_1, bu2_1)

    out_ref[0] = jnp.dot(h, W_o[...], preferred_element_type=jnp.float32) \
        + b_o[...]


def kernel(x, codes, anchors, W_x, b_x, W_c, b_c,
           Wm1_0, bm1_0, Wm2_0, bm2_0, Wu1_0, bu1_0, Wu2_0, bu2_0,
           Wm1_1, bm1_1, Wm2_1, bm2_1, Wu1_1, bu1_1, Wu2_1, bu2_1,
           W_o, b_o, interpret=False):
    B, N, _ = x.shape
    G = anchors.shape[0]
    H = W_x.shape[1]
    CH = W_o.shape[1]

    def split_m1(Wm1):
        return Wm1[:H], Wm1[H:2 * H], Wm1[2 * H:2 * H + 1]

    def split_u1(Wu1):
        return Wu1[:H], Wu1[H:]

    Wm1h_0, Wm1c_0, wd_0 = split_m1(Wm1_0)
    Wm1h_1, Wm1c_1, wd_1 = split_m1(Wm1_1)
    Wu1h_0, Wu1a_0 = split_u1(Wu1_0)
    Wu1h_1, Wu1a_1 = split_u1(Wu1_1)

    row = lambda v: v.reshape(1, -1)

    grid = (B, N // TILE)
    full = lambda shape: pl.BlockSpec(shape, lambda b, n: (0, 0))

    in_specs = [
        pl.BlockSpec((1, TILE, 3), lambda b, n: (b, n, 0)),
        pl.BlockSpec((1, G, codes.shape[-1]), lambda b, n: (b, 0, 0)),
        full((G, 3)),
        full(W_x.shape), full((1, H)),
        full(W_c.shape), full((1, H)),
    ]
    args = [x, codes, anchors, W_x, row(b_x), W_c, row(b_c)]
    for (Wm1h, Wm1c, wd, bm1, Wm2, bm2, Wu1h, Wu1a, bu1, Wu2, bu2) in (
        (Wm1h_0, Wm1c_0, wd_0, bm1_0, Wm2_0, bm2_0,
         Wu1h_0, Wu1a_0, bu1_0, Wu2_0, bu2_0),
        (Wm1h_1, Wm1c_1, wd_1, bm1_1, Wm2_1, bm2_1,
         Wu1h_1, Wu1a_1, bu1_1, Wu2_1, bu2_1),
    ):
        in_specs += [full((H, H)), full((H, H)), full((1, H)), full((1, H)),
                     full((H, H)), full((1, H)),
                     full((H, H)), full((H, H)), full((1, H)),
                     full((H, H)), full((1, H))]
        args += [Wm1h, Wm1c, wd, row(bm1), Wm2, row(bm2),
                 Wu1h, Wu1a, row(bu1), Wu2, row(bu2)]
    in_specs += [full((H, CH)), full((1, CH))]
    args += [W_o, row(b_o)]

    out = pl.pallas_call(
        _decoder_kernel,
        grid=grid,
        in_specs=in_specs,
        out_specs=pl.BlockSpec((1, TILE, CH), lambda b, n: (b, n, 0)),
        out_shape=jax.ShapeDtypeStruct((B, N, CH), jnp.float32),
        interpret=interpret,
    )(*args)
    return out
